# baseline (speedup 1.0000x reference)
import functools

import jax
import jax.numpy as jnp
from jax import lax
from jax.experimental import pallas as pl
from jax.experimental.pallas import tpu as pltpu
from jax.experimental.pallas import tpu_sc as plsc

_B = 16384
_D = 32
_L = 16
_NC = 2
_NS = 16
_NW = _NC * _NS
_BPW = _B // _NW
_GCH = 128
_NGATH = _BPW // _GCH

_mesh = plsc.VectorSubcoreMesh(core_axis_name="c", subcore_axis_name="s")


@functools.partial(
    pl.kernel,
    out_type=jax.ShapeDtypeStruct((_B,), jnp.float32),
    mesh=_mesh,
    scratch_types=[
        pltpu.VMEM((_BPW,), jnp.int32),
        pltpu.VMEM((_BPW,), jnp.int32),
        pltpu.VMEM((_BPW, _D), jnp.float32),
        pltpu.VMEM((_BPW, _D), jnp.float32),
        pltpu.VMEM((_BPW,), jnp.float32),
        pltpu.SemaphoreType.DMA,
    ],
    compiler_params=pltpu.CompilerParams(
        needs_layout_passes=False, use_tc_tiling_on_sc=False),
)
def _mf_sc(user_hbm, item_hbm, uf_hbm, if_hbm, out_hbm,
           uidx_v, iidx_v, urows_v, irows_v, out_v, sem):
    wid = lax.axis_index("s") * _NC + lax.axis_index("c")
    base = wid * _BPW

    pltpu.sync_copy(user_hbm.at[pl.ds(base, _BPW)], uidx_v)
    pltpu.sync_copy(item_hbm.at[pl.ds(base, _BPW)], iidx_v)

    def _sub1(i, carry):
        sl = pl.ds(i * _L, _L)
        uidx_v[sl] = uidx_v[sl] - 1
        iidx_v[sl] = iidx_v[sl] - 1
        return carry

    lax.fori_loop(0, _BPW // _L, _sub1, 0)

    copies = []
    for j in range(_NGATH):
        sl = pl.ds(j * _GCH, _GCH)
        copies.append(pltpu.async_copy(uf_hbm.at[uidx_v.at[sl]], urows_v.at[sl], sem))
        copies.append(pltpu.async_copy(if_hbm.at[iidx_v.at[sl]], irows_v.at[sl], sem))
    for c in copies:
        c.wait()

    def _grp(g, carry):
        row = g * _L + lax.iota(jnp.int32, _L)
        acc = jnp.zeros((_L,), jnp.float32)
        for d in range(_D):
            col = jnp.full((_L,), d, jnp.int32)
            gu = plsc.load_gather(urows_v, [row, col])
            gi = plsc.load_gather(irows_v, [row, col])
            acc = acc + gu * gi
        out_v[pl.ds(g * _L, _L)] = acc * 5.0
        return carry

    lax.fori_loop(0, _BPW // _L, _grp, 0)

    pltpu.sync_copy(out_v, out_hbm.at[pl.ds(base, _BPW)])


def kernel(user, item, user_factors, item_factors):
    return _mf_sc(user, item, user_factors, item_factors)


# speedup vs baseline: 1.5646x; 1.5646x over previous
import functools

import jax
import jax.numpy as jnp
from jax import lax
from jax.experimental import pallas as pl
from jax.experimental.pallas import tpu as pltpu
from jax.experimental.pallas import tpu_sc as plsc

_B = 16384
_D = 32
_LP = 128
_L = 16
_NC = 2
_NS = 16
_NW = _NC * _NS
_BPW = _B // _NW
_C = 128
_NCH = _BPW // _C

_mesh = plsc.VectorSubcoreMesh(core_axis_name="c", subcore_axis_name="s")


@functools.partial(
    pl.kernel,
    out_type=jax.ShapeDtypeStruct((_B,), jnp.float32),
    mesh=_mesh,
    scratch_types=[
        pltpu.VMEM((_BPW,), jnp.int32),
        pltpu.VMEM((_BPW,), jnp.int32),
        pltpu.VMEM((_C, _D), jnp.float32),
        pltpu.VMEM((_C, _D), jnp.float32),
        pltpu.VMEM((_C, _D), jnp.float32),
        pltpu.VMEM((_C, _D), jnp.float32),
        pltpu.VMEM((_BPW,), jnp.float32),
        pltpu.SemaphoreType.DMA,
        pltpu.SemaphoreType.DMA,
    ],
    compiler_params=pltpu.CompilerParams(
        needs_layout_passes=False, use_tc_tiling_on_sc=True),
)
def _mf_sc(user_hbm, item_hbm, uf_hbm, if_hbm, out_hbm,
           uidx_v, iidx_v,
           ubuf0, ubuf1, ibuf0, ibuf1, out_v, sem0, sem1):
    wid = lax.axis_index("s") * _NC + lax.axis_index("c")
    base = wid * _BPW

    pltpu.sync_copy(user_hbm.at[pl.ds(base, _BPW)], uidx_v)
    pltpu.sync_copy(item_hbm.at[pl.ds(base, _BPW)], iidx_v)

    ub = (ubuf0, ubuf1)
    ib = (ibuf0, ibuf1)
    sems = (sem0, sem1)

    def _fire(c, p):
        def body(v, carry):
            uvec = uidx_v[pl.ds(c * _C + v * _L, _L)] - 1
            ivec = iidx_v[pl.ds(c * _C + v * _L, _L)] - 1
            for j in range(_L):
                k = v * _L + j
                pltpu.async_copy(
                    uf_hbm.at[pl.ds(uvec[j], 1)], ub[p].at[pl.ds(k, 1)], sems[p])
                pltpu.async_copy(
                    if_hbm.at[pl.ds(ivec[j], 1)], ib[p].at[pl.ds(k, 1)], sems[p])
            return carry
        lax.fori_loop(0, _C // _L, body, 0)

    def _drain(p):
        pltpu.make_async_copy(uf_hbm.at[pl.ds(0, _C)], ub[p], sems[p]).wait()
        pltpu.make_async_copy(if_hbm.at[pl.ds(0, _C)], ib[p], sems[p]).wait()

    def _compute(c, p):
        u, it = ub[p], ib[p]

        def grp(g, carry):
            row = g * _L + lax.iota(jnp.int32, _L)
            acc = jnp.zeros((_L,), jnp.float32)
            for d in range(_D):
                col = jnp.full((_L,), d, jnp.int32)
                acc = acc + plsc.load_gather(u, [row, col]) * plsc.load_gather(it, [row, col])
            out_v[pl.ds(c * _C + g * _L, _L)] = acc * 5.0
            return carry

        lax.fori_loop(0, _C // _L, grp, 0)

    _fire(0, 0)
    for c in range(_NCH):
        p = c % 2
        if c + 1 < _NCH:
            _fire(c + 1, (c + 1) % 2)
        _drain(p)
        _compute(c, p)

    pltpu.sync_copy(out_v, out_hbm.at[pl.ds(base, _BPW)])


def kernel(user, item, user_factors, item_factors):
    return _mf_sc(user, item, user_factors, item_factors)
